# Initial kernel scaffold; baseline (speedup 1.0000x reference)
import jax
import jax.numpy as jnp
from jax.experimental import pallas as pl


def kernel(outputs, targets):
    raise NotImplementedError("write your pallas kernel here")


# speedup vs baseline: 5.5903x; 5.5903x over previous
import jax
import jax.numpy as jnp
from jax.experimental import pallas as pl
from jax.experimental.pallas import tpu as pltpu

K = 256
N_ROWS = 128
N_COLS = 2048


def _tc_body(x_ref, t_ref, out_ref):
    x = x_ref[:]
    t = t_ref[:]

    m = jnp.max(x, axis=1, keepdims=True)

    ub = jax.lax.bitcast_convert_type(x, jnp.uint32)
    ukey = jnp.where(ub >= jnp.uint32(0x80000000), ~ub,
                     ub | jnp.uint32(0x80000000))

    p = jnp.zeros((N_ROWS, 1), dtype=jnp.uint32)
    for i in range(31, -1, -1):
        cand = p | jnp.uint32(1 << i)
        cnt = jnp.sum((ukey >= cand).astype(jnp.int32), axis=1,
                      keepdims=True)
        p = jnp.where(cnt >= K, cand, p)

    ub_tau = jnp.where(p >= jnp.uint32(0x80000000),
                       p ^ jnp.uint32(0x80000000), ~p)
    tau = jax.lax.bitcast_convert_type(ub_tau, jnp.float32)

    gt = ukey > p
    c_gt = jnp.sum(gt.astype(jnp.int32), axis=1, keepdims=True)
    e = jnp.exp(x - m)
    s_above = jnp.sum(jnp.where(gt, e, 0.0), axis=1, keepdims=True)
    S = s_above + (K - c_gt).astype(jnp.float32) * jnp.exp(tau - m)

    col = jax.lax.broadcasted_iota(jnp.int32, (N_ROWS, N_COLS), 1)
    at_t = col == t
    v = jnp.sum(jnp.where(at_t, x, 0.0), axis=1, keepdims=True)
    ukey_i = jax.lax.bitcast_convert_type(ukey, jnp.int32)
    tu_i = jnp.sum(jnp.where(at_t, ukey_i, 0), axis=1, keepdims=True)
    tu = jax.lax.bitcast_convert_type(tu_i, jnp.uint32)

    eq_before = jnp.sum(((col < t) & (ukey == p)).astype(jnp.int32),
                        axis=1, keepdims=True)
    in_topk = (tu > p) | ((tu == p) & (c_gt + eq_before < K))
    inf = in_topk.astype(jnp.float32)

    contrib = v - m - jnp.log(S)
    total = jnp.sum(inf * contrib)
    count = jnp.sum(inf)
    out_ref[:, :] = jnp.full((1, 1), -(total / count), dtype=jnp.float32)


def kernel(outputs, targets):
    t32 = targets.astype(jnp.int32).reshape(N_ROWS, 1)
    out = pl.pallas_call(
        _tc_body,
        out_shape=jax.ShapeDtypeStruct((1, 1), jnp.float32),
    )(outputs, t32)
    return out.reshape(())
